# Initial kernel scaffold; baseline (speedup 1.0000x reference)
import jax
import jax.numpy as jnp
from jax.experimental import pallas as pl


def kernel(x, edge_index, W_self0, W_neigh0, b0, W_self1, W_neigh1, b1, W_self2, W_neigh2, b2):
    raise NotImplementedError("write your pallas kernel here")


# speedup vs baseline: 2.8234x; 2.8234x over previous
import functools

import jax
import jax.numpy as jnp
from jax import lax
from jax.experimental import pallas as pl
from jax.experimental.pallas import tpu as pltpu
from jax.experimental.pallas import tpu_sc as plsc

N = 10000
E = 320000
NC, NS = 2, 16
NW = NC * NS
CHUNK = 128
CPW = 80
GRP = 4
NGRP = CPW // GRP
EPW = CPW * CHUNK
EPAD = EPW * NW
R = 400
NPAD = 10112
RPS = NPAD // NS
NOUT = 12800
NBLK = N // R
PART1 = NOUT // R

f32 = jnp.float32


def _sc_agg(D, with_deg):
    mesh = plsc.VectorSubcoreMesh(
        core_axis_name="c", subcore_axis_name="s",
        num_cores=NC, num_subcores=NS)

    out_type = jax.ShapeDtypeStruct((NC * NOUT, D), f32)
    scratch = [
        pltpu.VMEM((GRP, CHUNK), jnp.int32),
        pltpu.VMEM((CHUNK, D), f32),
        pltpu.VMEM_SHARED((NPAD, D), f32),
    ]
    if not with_deg:
        scratch.insert(0, pltpu.VMEM((GRP, CHUNK), jnp.int32))

    def body(*refs):
        if with_deg:
            (onesrc, dstm, zrows, a_out, di, rows, acc) = refs
        else:
            (table, srcm, dstm, zrows, a_out, si, di, rows, acc) = refs
        c = lax.axis_index("c")
        s = lax.axis_index("s")
        wid = s * NC + c

        @pl.when(s == 0)
        def _():
            pltpu.sync_copy(zrows, acc)
        if with_deg:
            pltpu.sync_copy(onesrc, rows)
        plsc.subcore_barrier()

        def group(g, carry):
            base = wid * CPW + g * GRP
            if not with_deg:
                pltpu.sync_copy(srcm.at[pl.ds(base, GRP)], si)
            pltpu.sync_copy(dstm.at[pl.ds(base, GRP)], di)
            for j in range(GRP):
                if not with_deg:
                    pltpu.sync_copy(table.at[si.at[j]], rows)
                pltpu.sync_copy(rows, acc.at[di.at[j]], add=True)
            return carry

        lax.fori_loop(0, NGRP, group, 0)
        plsc.subcore_barrier()

        @pl.when(s == 0)
        def _():
            pltpu.sync_copy(acc, a_out.at[pl.ds(c * NOUT, NPAD)])

    return pl.kernel(body, out_type=out_type, mesh=mesh,
                     scratch_types=scratch)


def _tc_layer(din, dout, relu):
    def body(h, a0, a1, d0, d1, ws, wn, b, o):
        deg = d0[...][:, :1] + d1[...][:, :1]
        dinv = 1.0 / jnp.maximum(deg, 1.0)
        agg = (a0[...] + a1[...]) * dinv
        acc = jnp.dot(h[...], ws[...], preferred_element_type=f32)
        acc = acc + jnp.dot(agg, wn[...], preferred_element_type=f32)
        acc = acc + b[...]
        if relu:
            acc = jnp.maximum(acc, 0.0)
        o[...] = acc

    in_specs = [
        pl.BlockSpec((R, din), lambda i: (i, 0)),
        pl.BlockSpec((R, din), lambda i: (i, 0)),
        pl.BlockSpec((R, din), lambda i: (i + PART1, 0)),
        pl.BlockSpec((R, 128), lambda i: (i, 0)),
        pl.BlockSpec((R, 128), lambda i: (i + PART1, 0)),
        pl.BlockSpec((din, dout), lambda i: (0, 0)),
        pl.BlockSpec((din, dout), lambda i: (0, 0)),
        pl.BlockSpec((1, dout), lambda i: (0, 0)),
    ]

    return pl.pallas_call(
        body, grid=(NBLK,), in_specs=in_specs,
        out_specs=pl.BlockSpec((R, dout), lambda i: (i, 0)),
        out_shape=jax.ShapeDtypeStruct((N, dout), f32))


def kernel(x, edge_index, W_self0, W_neigh0, b0,
           W_self1, W_neigh1, b1, W_self2, W_neigh2, b2):
    src = edge_index[0]
    dst = edge_index[1]
    pad = EPAD - E
    srcm = jnp.concatenate(
        [src, jnp.zeros((pad,), jnp.int32)]).reshape(NW * CPW, CHUNK)
    dstm = jnp.concatenate(
        [dst, jnp.full((pad,), N, jnp.int32)]).reshape(NW * CPW, CHUNK)
    z128 = jnp.zeros((NPAD, 128), f32)
    ones128 = jnp.ones((CHUNK, 128), f32)

    b0r = b0.reshape(1, -1)
    b1r = b1.reshape(1, -1)
    b2r = b2.reshape(1, -1)

    degf = _sc_agg(128, True)(ones128, dstm, z128)
    a0 = _sc_agg(128, False)(x, srcm, dstm, z128)
    h1 = _tc_layer(128, 128, True)(
        x, a0, a0, degf, degf, W_self0, W_neigh0, b0r)
    a1 = _sc_agg(128, False)(h1, srcm, dstm, z128)
    h2 = _tc_layer(128, 128, True)(
        h1, a1, a1, degf, degf, W_self1, W_neigh1, b1r)
    a2 = _sc_agg(128, False)(h2, srcm, dstm, z128)
    out = _tc_layer(128, 64, False)(
        h2, a2, a2, degf, degf, W_self2, W_neigh2, b2r)
    return (out, h2)


# speedup vs baseline: 3.0869x; 1.0933x over previous
import functools

import jax
import jax.numpy as jnp
from jax import lax
from jax.experimental import pallas as pl
from jax.experimental.pallas import tpu as pltpu
from jax.experimental.pallas import tpu_sc as plsc

N = 10000
E = 320000
NC, NS = 2, 16
NW = NC * NS
CHUNK = 128
CPW = 80
GRP = 8
NGRP = CPW // GRP
EPW = CPW * CHUNK
EPAD = EPW * NW
R = 400
NPAD = 10112
RPS = NPAD // NS
NOUT = 12800
NBLK = N // R
PART1 = NOUT // R

f32 = jnp.float32


def _sc_agg(D, with_deg):
    mesh = plsc.VectorSubcoreMesh(
        core_axis_name="c", subcore_axis_name="s",
        num_cores=NC, num_subcores=NS)

    out_type = jax.ShapeDtypeStruct((NC * NOUT, D), f32)
    scratch = [
        pltpu.VMEM((GRP, CHUNK), jnp.int32),
        pltpu.VMEM((CHUNK, D), f32),
        pltpu.VMEM_SHARED((NPAD, D), f32),
    ]
    if not with_deg:
        scratch.insert(0, pltpu.VMEM((GRP, CHUNK), jnp.int32))
        scratch += [
            pltpu.VMEM((CHUNK, D), f32),
            pltpu.SemaphoreType.DMA,
            pltpu.SemaphoreType.DMA,
        ]

    def body(*refs):
        if with_deg:
            (onesrc, dstm, zrows, a_out, di, rows, acc) = refs
        else:
            (table, srcm, dstm, zrows, a_out,
             si, di, rowsa, acc, rowsb, sema, semb) = refs
            bufs = (rowsa, rowsb)
            sems = (sema, semb)
        c = lax.axis_index("c")
        s = lax.axis_index("s")
        wid = s * NC + c

        for k in range(NS):
            @pl.when(s == k)
            def _(k=k):
                pltpu.sync_copy(zrows.at[pl.ds(k * RPS, RPS)],
                                acc.at[pl.ds(k * RPS, RPS)])
        if with_deg:
            pltpu.sync_copy(onesrc, rows)
        plsc.subcore_barrier()

        if with_deg:
            def group(g, carry):
                base = wid * CPW + g * GRP
                pltpu.sync_copy(dstm.at[pl.ds(base, GRP)], di)
                for j in range(GRP):
                    pltpu.sync_copy(rows, acc.at[di.at[j]], add=True)
                return carry
        else:
            def group(g, carry):
                base = wid * CPW + g * GRP
                pltpu.sync_copy(srcm.at[pl.ds(base, GRP)], si)
                pltpu.sync_copy(dstm.at[pl.ds(base, GRP)], di)
                d = pltpu.async_copy(table.at[si.at[0]], bufs[0], sems[0])
                for j in range(GRP):
                    if j + 1 < GRP:
                        dn = pltpu.async_copy(
                            table.at[si.at[j + 1]],
                            bufs[(j + 1) % 2], sems[(j + 1) % 2])
                    d.wait()
                    pltpu.sync_copy(bufs[j % 2], acc.at[di.at[j]], add=True)
                    if j + 1 < GRP:
                        d = dn
                return carry

        lax.fori_loop(0, NGRP, group, 0)
        plsc.subcore_barrier()

        for k in range(NS):
            @pl.when(s == k)
            def _(k=k):
                pltpu.sync_copy(
                    acc.at[pl.ds(k * RPS, RPS)],
                    a_out.at[pl.ds(c * NOUT + k * RPS, RPS)])

    return pl.kernel(body, out_type=out_type, mesh=mesh,
                     scratch_types=scratch)


def _tc_layer(din, dout, relu):
    def body(h, a0, a1, d0, d1, ws, wn, b, o):
        deg = d0[...][:, :1] + d1[...][:, :1]
        dinv = 1.0 / jnp.maximum(deg, 1.0)
        agg = (a0[...] + a1[...]) * dinv
        acc = jnp.dot(h[...], ws[...], preferred_element_type=f32)
        acc = acc + jnp.dot(agg, wn[...], preferred_element_type=f32)
        acc = acc + b[...]
        if relu:
            acc = jnp.maximum(acc, 0.0)
        o[...] = acc

    in_specs = [
        pl.BlockSpec((R, din), lambda i: (i, 0)),
        pl.BlockSpec((R, din), lambda i: (i, 0)),
        pl.BlockSpec((R, din), lambda i: (i + PART1, 0)),
        pl.BlockSpec((R, 128), lambda i: (i, 0)),
        pl.BlockSpec((R, 128), lambda i: (i + PART1, 0)),
        pl.BlockSpec((din, dout), lambda i: (0, 0)),
        pl.BlockSpec((din, dout), lambda i: (0, 0)),
        pl.BlockSpec((1, dout), lambda i: (0, 0)),
    ]

    return pl.pallas_call(
        body, grid=(NBLK,), in_specs=in_specs,
        out_specs=pl.BlockSpec((R, dout), lambda i: (i, 0)),
        out_shape=jax.ShapeDtypeStruct((N, dout), f32))


def kernel(x, edge_index, W_self0, W_neigh0, b0,
           W_self1, W_neigh1, b1, W_self2, W_neigh2, b2):
    src = edge_index[0]
    dst = edge_index[1]
    pad = EPAD - E
    srcm = jnp.concatenate(
        [src, jnp.zeros((pad,), jnp.int32)]).reshape(NW * CPW, CHUNK)
    dstm = jnp.concatenate(
        [dst, jnp.full((pad,), N, jnp.int32)]).reshape(NW * CPW, CHUNK)
    z128 = jnp.zeros((NPAD, 128), f32)
    ones128 = jnp.ones((CHUNK, 128), f32)

    b0r = b0.reshape(1, -1)
    b1r = b1.reshape(1, -1)
    b2r = b2.reshape(1, -1)

    degf = _sc_agg(128, True)(ones128, dstm, z128)
    a0 = _sc_agg(128, False)(x, srcm, dstm, z128)
    h1 = _tc_layer(128, 128, True)(
        x, a0, a0, degf, degf, W_self0, W_neigh0, b0r)
    a1 = _sc_agg(128, False)(h1, srcm, dstm, z128)
    h2 = _tc_layer(128, 128, True)(
        h1, a1, a1, degf, degf, W_self1, W_neigh1, b1r)
    a2 = _sc_agg(128, False)(h2, srcm, dstm, z128)
    out = _tc_layer(128, 64, False)(
        h2, a2, a2, degf, degf, W_self2, W_neigh2, b2r)
    return (out, h2)


# speedup vs baseline: 3.3628x; 1.0894x over previous
import functools

import jax
import jax.numpy as jnp
from jax import lax
from jax.experimental import pallas as pl
from jax.experimental.pallas import tpu as pltpu
from jax.experimental.pallas import tpu_sc as plsc

N = 10000
E = 320000
NC, NS = 2, 16
NW = NC * NS
CHUNK = 64
CPW = 160
GRP = 16
NGRP = CPW // GRP
NBUF = 4
LOOK = NBUF - 1
DEGC = 128
EPW = CPW * CHUNK
EPAD = EPW * NW
R = 400
NPAD = 10112
RPS = NPAD // NS
NOUT = 12800
NBLK = N // R
PART1 = NOUT // R

f32 = jnp.float32


def _sc_agg(D, with_deg):
    mesh = plsc.VectorSubcoreMesh(
        core_axis_name="c", subcore_axis_name="s",
        num_cores=NC, num_subcores=NS)

    out_type = jax.ShapeDtypeStruct((NC * NOUT, D), f32)
    if with_deg:
        dgrp, dcpw = 8, EPW // DEGC
        scratch = [
            pltpu.VMEM((dgrp, DEGC), jnp.int32),
            pltpu.VMEM((DEGC, D), f32),
            pltpu.VMEM_SHARED((NPAD, D), f32),
        ]
    else:
        scratch = [
            pltpu.VMEM((GRP, CHUNK), jnp.int32),
            pltpu.VMEM((GRP, CHUNK), jnp.int32),
            pltpu.VMEM_SHARED((NPAD, D), f32),
        ]
        scratch += [pltpu.VMEM((CHUNK, D), f32) for _ in range(NBUF)]
        scratch += [pltpu.SemaphoreType.DMA for _ in range(NBUF)]

    def body(*refs):
        if with_deg:
            (onesrc, dstm, zrows, a_out, di, rows, acc) = refs
        else:
            (table, srcm, dstm, zrows, a_out, si, di, acc) = refs[:8]
            bufs = refs[8:8 + NBUF]
            sems = refs[8 + NBUF:8 + 2 * NBUF]
        c = lax.axis_index("c")
        s = lax.axis_index("s")
        wid = s * NC + c

        for k in range(NS):
            @pl.when(s == k)
            def _(k=k):
                pltpu.sync_copy(zrows.at[pl.ds(k * RPS, RPS)],
                                acc.at[pl.ds(k * RPS, RPS)])
        if with_deg:
            pltpu.sync_copy(onesrc, rows)
        plsc.subcore_barrier()

        if with_deg:
            def group(g, carry):
                base = wid * dcpw + g * dgrp
                pltpu.sync_copy(dstm.at[pl.ds(base, dgrp)], di)
                for j in range(dgrp):
                    pltpu.sync_copy(rows, acc.at[di.at[j]], add=True)
                return carry

            lax.fori_loop(0, dcpw // dgrp, group, 0)
        else:
            def group(g, carry):
                base = wid * CPW + g * GRP
                pltpu.sync_copy(srcm.at[pl.ds(base, GRP)], si)
                pltpu.sync_copy(dstm.at[pl.ds(base, GRP)], di)
                pend = [None] * GRP
                for j in range(LOOK):
                    pend[j] = pltpu.async_copy(
                        table.at[si.at[j]], bufs[j % NBUF], sems[j % NBUF])
                for j in range(GRP):
                    if j + LOOK < GRP:
                        pend[j + LOOK] = pltpu.async_copy(
                            table.at[si.at[j + LOOK]],
                            bufs[(j + LOOK) % NBUF], sems[(j + LOOK) % NBUF])
                    pend[j].wait()
                    pltpu.sync_copy(bufs[j % NBUF], acc.at[di.at[j]],
                                    add=True)
                return carry

            lax.fori_loop(0, NGRP, group, 0)
        plsc.subcore_barrier()

        for k in range(NS):
            @pl.when(s == k)
            def _(k=k):
                pltpu.sync_copy(
                    acc.at[pl.ds(k * RPS, RPS)],
                    a_out.at[pl.ds(c * NOUT + k * RPS, RPS)])

    return pl.kernel(body, out_type=out_type, mesh=mesh,
                     scratch_types=scratch)


def _tc_layer(din, dout, relu):
    def body(h, a0, a1, d0, d1, ws, wn, b, o):
        deg = d0[...][:, :1] + d1[...][:, :1]
        dinv = 1.0 / jnp.maximum(deg, 1.0)
        agg = (a0[...] + a1[...]) * dinv
        acc = jnp.dot(h[...], ws[...], preferred_element_type=f32)
        acc = acc + jnp.dot(agg, wn[...], preferred_element_type=f32)
        acc = acc + b[...]
        if relu:
            acc = jnp.maximum(acc, 0.0)
        o[...] = acc

    in_specs = [
        pl.BlockSpec((R, din), lambda i: (i, 0)),
        pl.BlockSpec((R, din), lambda i: (i, 0)),
        pl.BlockSpec((R, din), lambda i: (i + PART1, 0)),
        pl.BlockSpec((R, 128), lambda i: (i, 0)),
        pl.BlockSpec((R, 128), lambda i: (i + PART1, 0)),
        pl.BlockSpec((din, dout), lambda i: (0, 0)),
        pl.BlockSpec((din, dout), lambda i: (0, 0)),
        pl.BlockSpec((1, dout), lambda i: (0, 0)),
    ]

    return pl.pallas_call(
        body, grid=(NBLK,), in_specs=in_specs,
        out_specs=pl.BlockSpec((R, dout), lambda i: (i, 0)),
        out_shape=jax.ShapeDtypeStruct((N, dout), f32))


def kernel(x, edge_index, W_self0, W_neigh0, b0,
           W_self1, W_neigh1, b1, W_self2, W_neigh2, b2):
    src = edge_index[0]
    dst = edge_index[1]
    pad = EPAD - E
    srcf = jnp.concatenate([src, jnp.zeros((pad,), jnp.int32)])
    dstf = jnp.concatenate([dst, jnp.full((pad,), N, jnp.int32)])
    srcm = srcf.reshape(NW * CPW, CHUNK)
    dstm = dstf.reshape(NW * CPW, CHUNK)
    dstm_deg = dstf.reshape(EPAD // DEGC, DEGC)
    z128 = jnp.zeros((NPAD, 128), f32)
    ones128 = jnp.ones((DEGC, 128), f32)

    b0r = b0.reshape(1, -1)
    b1r = b1.reshape(1, -1)
    b2r = b2.reshape(1, -1)

    degf = _sc_agg(128, True)(ones128, dstm_deg, z128)
    a0 = _sc_agg(128, False)(x, srcm, dstm, z128)
    h1 = _tc_layer(128, 128, True)(
        x, a0, a0, degf, degf, W_self0, W_neigh0, b0r)
    a1 = _sc_agg(128, False)(h1, srcm, dstm, z128)
    h2 = _tc_layer(128, 128, True)(
        h1, a1, a1, degf, degf, W_self1, W_neigh1, b1r)
    a2 = _sc_agg(128, False)(h2, srcm, dstm, z128)
    out = _tc_layer(128, 64, False)(
        h2, a2, a2, degf, degf, W_self2, W_neigh2, b2r)
    return (out, h2)
